# baseline (speedup 1.0000x reference)
import functools

import jax
import jax.numpy as jnp
from jax import lax
from jax.experimental import pallas as pl
from jax.experimental.pallas import tpu as pltpu
from jax.experimental.pallas import tpu_sc as plsc

N = 32768
K = 1024
D = 64
R = 1024
NBLK = N // R

NC = 2
NS = 16
NW = NC * NS
BPW = N // NW
CH = 128
NCH = BPW // CH

_SCALE = (1.0 + 0.25) / float(N * D)


def _tc_body(x_ref, e_ref, idx_ref, loss_ref, e2_ref, b2_ref, ks_ref):
    i = pl.program_id(0)

    @pl.when(i == 0)
    def _once():
        e = e_ref[...]
        e2_ref[...] = -2.0 * e
        b2_ref[...] = jnp.sum(e * e, axis=1, keepdims=True)
        ks_ref[...] = lax.broadcasted_iota(
            jnp.int32, (K, 1), 0).astype(jnp.float32)
        loss_ref[...] = jnp.zeros((1, 1), jnp.float32)

    x = x_ref[...]
    a2 = jnp.sum(x * x, axis=1, keepdims=True)
    a2r = lax.transpose(a2, (1, 0))
    KC = 16
    mm2 = lax.dot_general(e2_ref[...], x, (((1,), (1,)), ((), ())),
                          preferred_element_type=jnp.float32)
    run_min = None
    run_idx = None
    for c in range(K // KC):
        b2c = b2_ref[pl.ds(c * KC, KC), :]
        ksc = ks_ref[pl.ds(c * KC, KC), :]
        s = (a2r + b2c) + mm2[c * KC:(c + 1) * KC, :]
        if run_min is None:
            run_min = s
            run_idx = jnp.broadcast_to(ksc, (KC, R))
        else:
            m = s < run_min
            run_min = jnp.minimum(s, run_min)
            run_idx = jnp.where(m, ksc, run_idx)
    mins = jnp.min(run_min, axis=0, keepdims=True)
    idx_f = jnp.min(jnp.where(run_min == mins, run_idx, float(K)), axis=0)
    idx_ref[0, :, :] = idx_f.astype(jnp.int32).reshape(NCH, CH)

    loss_ref[...] += jnp.sum(mins, axis=1, keepdims=True)

    @pl.when(i == NBLK - 1)
    def _finish():
        loss_ref[...] = loss_ref[...] * _SCALE


def _tc_quantize(flat, embeddings, interpret=False):
    return pl.pallas_call(
        _tc_body,
        grid=(NBLK,),
        in_specs=[
            pl.BlockSpec((R, D), lambda i: (i, 0)),
            pl.BlockSpec((K, D), lambda i: (0, 0)),
        ],
        out_specs=[
            pl.BlockSpec((1, NCH, CH), lambda i: (i, 0, 0)),
            pl.BlockSpec((1, 1), lambda i: (0, 0)),
        ],
        out_shape=[
            jax.ShapeDtypeStruct((NW, NCH, CH), jnp.int32),
            jax.ShapeDtypeStruct((1, 1), jnp.float32),
        ],
        scratch_shapes=[pltpu.VMEM((K, D), jnp.float32),
                        pltpu.VMEM((K, 1), jnp.float32),
                        pltpu.VMEM((K, 1), jnp.float32)],
        interpret=interpret,
    )(flat, embeddings)


NBUF = 4


def _sc_gather_body(idx_hbm, table_hbm, out_hbm, idx_v, bufs, sem):
    wid = lax.axis_index("s") * NC + lax.axis_index("c")
    pltpu.sync_copy(idx_hbm.at[wid], idx_v)
    copies = [None] * NCH
    for j in range(NBUF):
        copies[j] = pltpu.async_copy(table_hbm.at[idx_v.at[j]],
                                     bufs.at[j], sem)
    for j in range(NCH):
        copies[j].wait()
        pltpu.sync_copy(bufs.at[j % NBUF],
                        out_hbm.at[wid, pl.ds(j * CH, CH)])
        if j + NBUF < NCH:
            copies[j + NBUF] = pltpu.async_copy(
                table_hbm.at[idx_v.at[j + NBUF]], bufs.at[j % NBUF], sem)


@functools.lru_cache(maxsize=1)
def _sc_gather_kernel():
    return pl.kernel(
        _sc_gather_body,
        mesh=plsc.VectorSubcoreMesh(core_axis_name="c", subcore_axis_name="s"),
        out_type=jax.ShapeDtypeStruct((NW, BPW, D), jnp.float32),
        scratch_types=[
            pltpu.VMEM((NCH, CH), jnp.int32),
            pltpu.VMEM((NBUF, CH, D), jnp.float32),
            pltpu.SemaphoreType.DMA,
        ],
        compiler_params=pltpu.CompilerParams(use_tc_tiling_on_sc=False),
    )


def kernel(inputs, embeddings):
    flat = inputs.reshape(N, D)
    idx, loss = _tc_quantize(flat, embeddings)
    quantized = _sc_gather_kernel()(idx, embeddings)
    return quantized.reshape(inputs.shape), loss[0, 0]


# speedup vs baseline: 1.0932x; 1.0932x over previous
import functools

import jax
import jax.numpy as jnp
from jax import lax
from jax.experimental import pallas as pl
from jax.experimental.pallas import tpu as pltpu
from jax.experimental.pallas import tpu_sc as plsc

N = 32768
K = 1024
D = 64
R = 2048
NBLK = N // R

NC = 2
NS = 16
NW = NC * NS
BPW = N // NW
CH = 128
NCH = BPW // CH

_SCALE = (1.0 + 0.25) / float(N * D)


def _tc_body(x_ref, e_ref, idx_ref, loss_ref, e2_ref, b2_ref, ks_ref):
    i = pl.program_id(0)

    @pl.when(i == 0)
    def _once():
        e = e_ref[...]
        e2_ref[...] = -2.0 * e
        b2_ref[...] = jnp.sum(e * e, axis=1, keepdims=True)
        ks_ref[...] = lax.broadcasted_iota(
            jnp.int32, (K, 1), 0).astype(jnp.float32)
        loss_ref[...] = jnp.zeros((1, 1), jnp.float32)

    x = x_ref[...]
    a2 = jnp.sum(x * x, axis=1, keepdims=True)
    a2r = lax.transpose(a2, (1, 0))
    KC = 16
    mm2 = lax.dot_general(e2_ref[...], x, (((1,), (1,)), ((), ())),
                          preferred_element_type=jnp.float32)
    run_min = None
    run_idx = None
    for c in range(K // KC):
        b2c = b2_ref[pl.ds(c * KC, KC), :]
        ksc = ks_ref[pl.ds(c * KC, KC), :]
        s = (a2r + b2c) + mm2[c * KC:(c + 1) * KC, :]
        if run_min is None:
            run_min = s
            run_idx = jnp.broadcast_to(ksc, (KC, R))
        else:
            m = s < run_min
            run_min = jnp.minimum(s, run_min)
            run_idx = jnp.where(m, ksc, run_idx)
    mins = jnp.min(run_min, axis=0, keepdims=True)
    idx_f = jnp.min(jnp.where(run_min == mins, run_idx, float(K)), axis=0)
    idx_ref[...] = idx_f.astype(jnp.int32).reshape(R // BPW, NCH, CH)

    loss_ref[...] += jnp.sum(mins, axis=1, keepdims=True)

    @pl.when(i == NBLK - 1)
    def _finish():
        loss_ref[...] = loss_ref[...] * _SCALE


def _tc_quantize(flat, embeddings, interpret=False):
    return pl.pallas_call(
        _tc_body,
        grid=(NBLK,),
        in_specs=[
            pl.BlockSpec((R, D), lambda i: (i, 0)),
            pl.BlockSpec((K, D), lambda i: (0, 0)),
        ],
        out_specs=[
            pl.BlockSpec((R // 1024, NCH, CH), lambda i: (i, 0, 0)),
            pl.BlockSpec((1, 1), lambda i: (0, 0)),
        ],
        out_shape=[
            jax.ShapeDtypeStruct((NW, NCH, CH), jnp.int32),
            jax.ShapeDtypeStruct((1, 1), jnp.float32),
        ],
        scratch_shapes=[pltpu.VMEM((K, D), jnp.float32),
                        pltpu.VMEM((K, 1), jnp.float32),
                        pltpu.VMEM((K, 1), jnp.float32)],
        interpret=interpret,
    )(flat, embeddings)


NBUF = 4


def _sc_gather_body(idx_hbm, table_hbm, out_hbm, idx_v, bufs, sem):
    wid = lax.axis_index("s") * NC + lax.axis_index("c")
    pltpu.sync_copy(idx_hbm.at[wid], idx_v)
    copies = [None] * NCH
    for j in range(NBUF):
        copies[j] = pltpu.async_copy(table_hbm.at[idx_v.at[j]],
                                     bufs.at[j], sem)
    for j in range(NCH):
        copies[j].wait()
        pltpu.sync_copy(bufs.at[j % NBUF],
                        out_hbm.at[wid, pl.ds(j * CH, CH)])
        if j + NBUF < NCH:
            copies[j + NBUF] = pltpu.async_copy(
                table_hbm.at[idx_v.at[j + NBUF]], bufs.at[j % NBUF], sem)


@functools.lru_cache(maxsize=1)
def _sc_gather_kernel():
    return pl.kernel(
        _sc_gather_body,
        mesh=plsc.VectorSubcoreMesh(core_axis_name="c", subcore_axis_name="s"),
        out_type=jax.ShapeDtypeStruct((NW, BPW, 2 * D), jnp.float32),
        scratch_types=[
            pltpu.VMEM((NCH, CH), jnp.int32),
            pltpu.VMEM((NBUF, CH, 2 * D), jnp.float32),
            pltpu.SemaphoreType.DMA,
        ],
        compiler_params=pltpu.CompilerParams(use_tc_tiling_on_sc=True),
    )


def kernel(inputs, embeddings):
    flat = inputs.reshape(N, D)
    idx, loss = _tc_quantize(flat, embeddings)
    table = jnp.pad(embeddings, ((0, 0), (0, D)))
    quantized = _sc_gather_kernel()(idx, table)
    quantized = quantized[:, :, :D]
    return quantized.reshape(inputs.shape), loss[0, 0]
